# baseline (speedup 1.0000x reference)
import functools

import jax
import jax.numpy as jnp
from jax import lax
from jax.experimental import pallas as pl
from jax.experimental.pallas import tpu as pltpu, tpu_sc as plsc

_NUM_STYLES = 1000
_DIM = 64
_BATCH = 16384

_NC = 2
_NS = 16
_NW = _NC * _NS
_BPW = _BATCH // _NW
_CHUNK = 64
_NCHUNK = _BPW // _CHUNK


def _emb_body(idx_hbm, table_hbm, out_hbm, table_s, idx_v, rows_v, gsem, osem):
    cid = lax.axis_index("c")
    sid = lax.axis_index("s")
    base = (sid * _NC + cid) * _BPW

    @pl.when(sid == 0)
    def _stage_table():
        pltpu.sync_copy(table_hbm, table_s)

    pltpu.sync_copy(idx_hbm.at[pl.ds(base, _BPW)], idx_v)
    plsc.subcore_barrier()

    @pl.loop(0, _NCHUNK)
    def _gather(j):
        pltpu.async_copy(
            table_s.at[idx_v.at[pl.ds(j * _CHUNK, _CHUNK)]],
            rows_v.at[pl.ds(j * _CHUNK, _CHUNK)],
            gsem,
        )

    @pl.loop(0, _NCHUNK)
    def _drain(j):
        pltpu.make_async_copy(
            table_s.at[idx_v.at[pl.ds(j * _CHUNK, _CHUNK)]],
            rows_v.at[pl.ds(j * _CHUNK, _CHUNK)],
            gsem,
        ).wait()
        pltpu.async_copy(
            rows_v.at[pl.ds(j * _CHUNK, _CHUNK)],
            out_hbm.at[pl.ds(base + j * _CHUNK, _CHUNK), pl.ds(0, _DIM)],
            osem,
        )

    @pl.loop(0, _NCHUNK)
    def _finish(j):
        pltpu.make_async_copy(
            rows_v.at[pl.ds(j * _CHUNK, _CHUNK)],
            out_hbm.at[pl.ds(base + j * _CHUNK, _CHUNK), pl.ds(0, _DIM)],
            osem,
        ).wait()


_emb = functools.partial(
    pl.kernel,
    out_type=jax.ShapeDtypeStruct((_BATCH, 2 * _DIM), jnp.float32),
    mesh=plsc.VectorSubcoreMesh(core_axis_name="c", subcore_axis_name="s"),
    scratch_types=[
        pltpu.VMEM_SHARED((_NUM_STYLES, _DIM), jnp.float32),
        pltpu.VMEM((_BPW,), jnp.int32),
        pltpu.VMEM((_BPW, _DIM), jnp.float32),
        pltpu.SemaphoreType.DMA,
        pltpu.SemaphoreType.DMA,
    ],
    compiler_params=pltpu.CompilerParams(
        use_tc_tiling_on_sc=False, needs_layout_passes=False
    ),
)(_emb_body)


def kernel(style_id, embed_weight):
    padded = _emb(style_id.astype(jnp.int32), embed_weight)
    return lax.slice(padded, (0, 0), (_BATCH, _DIM))
